# baseline (speedup 1.0000x reference)
import functools

import jax
import jax.numpy as jnp
from jax import lax
from jax.experimental import pallas as pl
from jax.experimental.pallas import tpu as pltpu
from jax.experimental.pallas import tpu_sc as plsc

VOCAB = 100001
EMBED_DIM = 32
DENSE_OUT = 32
BATCH = 16384

_VBLK = 256
_VGRID = 98
_WIDE = _VGRID * _VBLK


def _project_pack_tc(table, W, b2d):
    def body(t0, t1, t2, t3, w_ref, b_ref, o_ref):
        w = w_ref[...]
        bb = b_ref[...]
        parts = [
            jnp.dot(t[...], w, preferred_element_type=jnp.float32) + bb
            for t in (t0, t1, t2, t3)
        ]
        o_ref[...] = jnp.concatenate(parts, axis=1)

    n_blocks = VOCAB // _VBLK

    def t_spec(j):
        return pl.BlockSpec(
            (_VBLK, EMBED_DIM),
            lambda i: (jnp.minimum(i + j * _VGRID, n_blocks), 0),
        )

    return pl.pallas_call(
        body,
        grid=(_VGRID,),
        in_specs=[
            t_spec(0),
            t_spec(1),
            t_spec(2),
            t_spec(3),
            pl.BlockSpec((EMBED_DIM, DENSE_OUT), lambda i: (0, 0)),
            pl.BlockSpec((1, DENSE_OUT), lambda i: (0, 0)),
        ],
        out_specs=pl.BlockSpec((_VBLK, 128), lambda i: (i, 0)),
        out_shape=jax.ShapeDtypeStruct((_WIDE, 128), jnp.float32),
    )(table, table, table, table, W, b2d)


def _gather_sc(tw, idxw):
    info = plsc.get_sparse_core_info()
    nc, ns = info.num_cores, info.num_subcores
    nw = nc * ns
    b_per_w = BATCH // nw
    mesh = plsc.VectorSubcoreMesh(core_axis_name="c", subcore_axis_name="s")

    @functools.partial(
        pl.kernel,
        mesh=mesh,
        out_type=jax.ShapeDtypeStruct((BATCH, 128), jnp.float32),
        scratch_types=[
            pltpu.VMEM((b_per_w,), jnp.int32),
            pltpu.VMEM((b_per_w, 128), jnp.float32),
            pltpu.SemaphoreType.DMA,
        ],
    )
    def gather_kernel(tw_hbm, idx_hbm, out_hbm, idx_v, rows_v, sem):
        wid = lax.axis_index("s") * nc + lax.axis_index("c")
        base = wid * b_per_w
        pltpu.sync_copy(idx_hbm.at[pl.ds(base, b_per_w)], idx_v)
        pltpu.async_copy(tw_hbm.at[idx_v], rows_v, sem).wait()
        pltpu.sync_copy(rows_v, out_hbm.at[pl.ds(base, b_per_w)])

    return gather_kernel(tw, idxw)


def _select_tc(g, sel2d):
    blk = 2048

    def body(g_ref, s_ref, o_ref):
        gv = g_ref[...]
        s = s_ref[...]
        o_ref[...] = jnp.where(
            s == 0,
            gv[:, 0:32],
            jnp.where(
                s == 1,
                gv[:, 32:64],
                jnp.where(s == 2, gv[:, 64:96], gv[:, 96:128]),
            ),
        )

    return pl.pallas_call(
        body,
        grid=(BATCH // blk,),
        in_specs=[
            pl.BlockSpec((blk, 128), lambda i: (i, 0)),
            pl.BlockSpec((blk, 1), lambda i: (i, 0)),
        ],
        out_specs=pl.BlockSpec((blk, DENSE_OUT), lambda i: (i, 0)),
        out_shape=jax.ShapeDtypeStruct((BATCH, DENSE_OUT), jnp.float32),
    )(g, sel2d)


def kernel(user_id, table, W, b):
    uid = user_id.astype(jnp.int32)
    tw = _project_pack_tc(table, W, b.reshape(1, DENSE_OUT))
    g = _gather_sc(tw, uid % _WIDE)
    return _select_tc(g, (uid // _WIDE).reshape(BATCH, 1))


# speedup vs baseline: 1.7707x; 1.7707x over previous
import functools

import jax
import jax.numpy as jnp
from jax import lax
from jax.experimental import pallas as pl
from jax.experimental.pallas import tpu as pltpu
from jax.experimental.pallas import tpu_sc as plsc

VOCAB = 100001
EMBED_DIM = 32
DENSE_OUT = 32
BATCH = 16384
_GROUPS = 4
_GROUP_ROWS = BATCH // _GROUPS


def _gather_pack_sc(table, idx):
    info = plsc.get_sparse_core_info()
    nc, ns = info.num_cores, info.num_subcores
    nw = nc * ns
    b_per_w = BATCH // nw
    w_per_grp = _GROUP_ROWS // b_per_w
    mesh = plsc.VectorSubcoreMesh(core_axis_name="c", subcore_axis_name="s")

    @functools.partial(
        pl.kernel,
        mesh=mesh,
        out_type=jax.ShapeDtypeStruct((_GROUP_ROWS, 128), jnp.float32),
        scratch_types=[
            pltpu.VMEM((b_per_w,), jnp.int32),
            pltpu.VMEM((b_per_w, EMBED_DIM), jnp.float32),
            pltpu.SemaphoreType.DMA,
        ],
        compiler_params=pltpu.CompilerParams(use_tc_tiling_on_sc=False),
    )
    def gather_kernel(table_hbm, idx_hbm, out_hbm, idx_v, rows_v, sem):
        wid = lax.axis_index("s") * nc + lax.axis_index("c")
        base = wid * b_per_w
        pltpu.sync_copy(idx_hbm.at[pl.ds(base, b_per_w)], idx_v)
        pltpu.async_copy(table_hbm.at[idx_v], rows_v, sem).wait()
        grp = wid // w_per_grp
        row0 = (wid % w_per_grp) * b_per_w
        pltpu.sync_copy(
            rows_v,
            out_hbm.at[pl.ds(row0, b_per_w), pl.ds(grp * EMBED_DIM, EMBED_DIM)],
        )

    return gather_kernel(table, idx)


def _project_tc(g, W, b2d):
    def body(g_ref, w_ref, b_ref, o_ref):
        j = pl.program_id(0)
        gv = g_ref[...]
        e = jnp.where(
            j == 0,
            gv[:, 0:32],
            jnp.where(
                j == 1,
                gv[:, 32:64],
                jnp.where(j == 2, gv[:, 64:96], gv[:, 96:128]),
            ),
        )
        o_ref[...] = (
            jnp.dot(e, w_ref[...], preferred_element_type=jnp.float32)
            + b_ref[...]
        )

    return pl.pallas_call(
        body,
        grid=(_GROUPS,),
        in_specs=[
            pl.BlockSpec((_GROUP_ROWS, 128), lambda j: (0, 0)),
            pl.BlockSpec((EMBED_DIM, DENSE_OUT), lambda j: (0, 0)),
            pl.BlockSpec((1, DENSE_OUT), lambda j: (0, 0)),
        ],
        out_specs=pl.BlockSpec((_GROUP_ROWS, DENSE_OUT), lambda j: (j, 0)),
        out_shape=jax.ShapeDtypeStruct((BATCH, DENSE_OUT), jnp.float32),
    )(g, W, b2d)


def kernel(user_id, table, W, b):
    uid = user_id.astype(jnp.int32)
    g = _gather_pack_sc(table, uid)
    return _project_tc(g, W, b.reshape(1, DENSE_OUT))
